# baseline (device time: 13148 ns/iter reference)
import jax
import jax.numpy as jnp
from jax import lax
from jax.experimental import pallas as pl
from jax.experimental.pallas import tpu as pltpu

M = 1024
D = 512
HALF = M // 2
K = 4
CH = HALF // K


def kernel(partial, gamma):
    x = partial.reshape(M, D)
    g = gamma.reshape(1, D)

    def body(
        x_ref,
        g_ref,
        out_ref,
        mine_ref,
        peer_ref,
        send_ref,
        recv_ref,
        copy_sems,
        send_sems,
        recv_sems,
    ):
        my_x = lax.axis_index("x")
        my_y = lax.axis_index("y")
        my_z = lax.axis_index("z")
        peer = (my_x, my_y, 1 - my_z)
        my_base = my_z * HALF
        peer_base = (1 - my_z) * HALF

        mine_copy = pltpu.make_async_copy(
            x_ref.at[pl.ds(my_base, HALF), :], mine_ref, copy_sems.at[K]
        )
        mine_copy.start()
        peer_copies = []
        for k in range(K):
            c = pltpu.make_async_copy(
                x_ref.at[pl.ds(peer_base + k * CH, CH), :],
                peer_ref.at[k],
                copy_sems.at[k],
            )
            c.start()
            peer_copies.append(c)

        barrier_sem = pltpu.get_barrier_semaphore()
        pl.semaphore_signal(
            barrier_sem,
            inc=1,
            device_id=peer,
            device_id_type=pl.DeviceIdType.MESH,
        )
        pl.semaphore_wait(barrier_sem, 1)

        rdmas = []
        for k in range(K):
            peer_copies[k].wait()
            send_ref[k] = peer_ref[k].astype(jnp.bfloat16)
            r = pltpu.make_async_remote_copy(
                src_ref=send_ref.at[k],
                dst_ref=recv_ref.at[k],
                send_sem=send_sems.at[k],
                recv_sem=recv_sems.at[k],
                device_id=peer,
                device_id_type=pl.DeviceIdType.MESH,
            )
            r.start()
            rdmas.append(r)

        mine_copy.wait()
        for k in range(K):
            rdmas[k].wait_recv()
            y = mine_ref[pl.ds(k * CH, CH), :] + recv_ref[k].astype(
                jnp.float32
            )
            ms = jnp.mean(y * y, axis=-1, keepdims=True)
            out_ref[pl.ds(k * CH, CH), :] = y * lax.rsqrt(ms + 1e-6) * g_ref[...]

        for k in range(K):
            rdmas[k].wait_send()

    return pl.pallas_call(
        body,
        out_shape=jax.ShapeDtypeStruct((HALF, D), jnp.float32),
        in_specs=[
            pl.BlockSpec(memory_space=pl.ANY),
            pl.BlockSpec(memory_space=pltpu.VMEM),
        ],
        out_specs=pl.BlockSpec(memory_space=pltpu.VMEM),
        scratch_shapes=[
            pltpu.VMEM((HALF, D), jnp.float32),
            pltpu.VMEM((K, CH, D), jnp.float32),
            pltpu.VMEM((K, CH, D), jnp.bfloat16),
            pltpu.VMEM((K, CH, D), jnp.bfloat16),
            pltpu.SemaphoreType.DMA((K + 1,)),
            pltpu.SemaphoreType.DMA((K,)),
            pltpu.SemaphoreType.DMA((K,)),
        ],
        compiler_params=pltpu.CompilerParams(collective_id=0),
    )(x, g)
